# baseline (speedup 1.0000x reference)
import functools

import jax
import jax.numpy as jnp
from jax import lax
from jax.experimental import pallas as pl
from jax.experimental.pallas import tpu as pltpu
from jax.experimental.pallas import tpu_sc as plsc

N = 10000
E = 320000
D = 128
DW = 8

NC = 2
NS = 16
NW = NC * NS
EPW = E // NW
B = 50
NB = EPW // B
NCH = 2
NB2 = NB // NCH
NBUF = 4
RPT = N // NS

_mesh = plsc.VectorSubcoreMesh(core_axis_name="c", subcore_axis_name="s")


@functools.partial(
    pl.kernel,
    out_type=(
        jax.ShapeDtypeStruct((NC, N, D), jnp.float32),
        jax.ShapeDtypeStruct((NC, N, DW), jnp.float32),
    ),
    mesh=_mesh,
    compiler_params=pltpu.CompilerParams(use_tc_tiling_on_sc=False),
    scratch_types=[
        pltpu.VMEM((NB2, B), jnp.int32),
        pltpu.VMEM((NB2, B), jnp.int32),
        pltpu.VMEM((B, D), jnp.float32),
        pltpu.VMEM((B, D), jnp.float32),
        pltpu.VMEM((B, D), jnp.float32),
        pltpu.VMEM((B, D), jnp.float32),
        pltpu.VMEM((B, DW), jnp.float32),
        pltpu.VMEM_SHARED((N, D), jnp.float32),
        pltpu.VMEM_SHARED((N, DW), jnp.float32),
        pltpu.SemaphoreType.DMA,
        pltpu.SemaphoreType.DMA,
        pltpu.SemaphoreType.DMA,
    ],
)
def _sc_segment_sum(feat_hbm, src_hbm, dst_hbm, zrows_hbm, zdeg_hbm, ones_hbm,
                    agg_out, deg_out,
                    src_v, dst_v, r0, r1, r2, r3, ones_v, agg_sh, deg_sh,
                    gsem, ssem, dsem):
    c = lax.axis_index("c")
    s = lax.axis_index("s")
    ring = (r0, r1, r2, r3)

    pltpu.sync_copy(ones_hbm, ones_v)

    pltpu.sync_copy(zrows_hbm, agg_sh.at[pl.ds(s * RPT, RPT)])
    pltpu.sync_copy(zdeg_hbm, deg_sh.at[pl.ds(s * RPT, RPT)])
    plsc.subcore_barrier()

    def gather(j, buf):
        pltpu.async_copy(feat_hbm.at[src_v.at[j]], buf, gsem)

    def gather_wait(j, buf):
        pltpu.make_async_copy(feat_hbm.at[src_v.at[j]], buf, gsem).wait()

    def scatter(j, buf):
        pltpu.async_copy(buf, agg_sh.at[dst_v.at[j]], ssem, add=True)
        pltpu.async_copy(ones_v, deg_sh.at[dst_v.at[j]], dsem, add=True)

    def scatter_wait(j, buf):
        pltpu.make_async_copy(buf, agg_sh.at[dst_v.at[j]], ssem).wait()

    for ch in range(NCH):
        pltpu.sync_copy(src_hbm.at[c, s, ch], src_v)
        pltpu.sync_copy(dst_hbm.at[c, s, ch], dst_v)

        for b in range(NBUF):
            gather(b, ring[b])

        def body(i, carry):
            j = NBUF * i
            for b in range(NBUF):
                gather_wait(j + b, ring[b])
                scatter(j + b, ring[b])
            for b in range(NBUF):
                scatter_wait(j + b, ring[b])
                gather(j + NBUF + b, ring[b])
            return carry

        lax.fori_loop(0, NB2 // NBUF - 1, body, 0)

        jlast = NB2 - NBUF
        for b in range(NBUF):
            gather_wait(jlast + b, ring[b])
            scatter(jlast + b, ring[b])
        for b in range(NBUF):
            scatter_wait(jlast + b, ring[b])

        pltpu.make_async_copy(deg_out.at[c, pl.ds(0, NB2 * B)],
                              deg_sh.at[pl.ds(0, NB2 * B)], dsem).wait()
    plsc.subcore_barrier()

    pltpu.sync_copy(agg_sh.at[pl.ds(s * RPT, RPT)],
                    agg_out.at[c, pl.ds(s * RPT, RPT)])
    pltpu.sync_copy(deg_sh.at[pl.ds(s * RPT, RPT)],
                    deg_out.at[c, pl.ds(s * RPT, RPT)])


_R = 1000


def _tc_layer_body(relu, x_ref, agg_ref, deg_ref, ws_ref, wn_ref, b_ref,
                   o_ref):
    deg = deg_ref[0, :, 0] + deg_ref[1, :, 0]
    mean = (agg_ref[0] + agg_ref[1]) / jnp.maximum(deg, 1.0)[:, None]
    acc = jnp.dot(x_ref[...], ws_ref[...],
                  preferred_element_type=jnp.float32,
                  precision=lax.Precision.HIGHEST)
    acc = acc + jnp.dot(mean, wn_ref[...],
                        preferred_element_type=jnp.float32,
                        precision=lax.Precision.HIGHEST)
    acc = acc + b_ref[...]
    if relu:
        acc = jnp.maximum(acc, 0.0)
    o_ref[...] = acc


def _tc_sage_layer(x, agg, deg, W_self, W_neigh, b, relu):
    h = W_self.shape[1]
    return pl.pallas_call(
        functools.partial(_tc_layer_body, relu),
        grid=(N // _R,),
        in_specs=[
            pl.BlockSpec((_R, D), lambda i: (i, 0)),
            pl.BlockSpec((NC, _R, D), lambda i: (0, i, 0)),
            pl.BlockSpec((NC, _R, DW), lambda i: (0, i, 0)),
            pl.BlockSpec((D, h), lambda i: (0, 0)),
            pl.BlockSpec((D, h), lambda i: (0, 0)),
            pl.BlockSpec((1, h), lambda i: (0, 0)),
        ],
        out_specs=pl.BlockSpec((_R, h), lambda i: (i, 0)),
        out_shape=jax.ShapeDtypeStruct((N, h), jnp.float32),
    )(x, agg, deg, W_self, W_neigh, b.reshape(1, h))


def kernel(x, edge_index1, edge_index2, W_self1, W_neigh1, b1,
           W_self2, W_neigh2, b2):
    zrows = jnp.zeros((RPT, D), jnp.float32)
    zdeg = jnp.zeros((RPT, DW), jnp.float32)
    ones = jnp.ones((B, DW), jnp.float32)

    def edges(ei):
        src = ei[0].astype(jnp.int32).reshape(NC, NS, NCH, NB2, B)
        dst = ei[1].astype(jnp.int32).reshape(NC, NS, NCH, NB2, B)
        return src, dst

    src1, dst1 = edges(edge_index1)
    src2, dst2 = edges(edge_index2)

    agg1, deg1 = _sc_segment_sum(x, src1, dst1, zrows, zdeg, ones)
    h = _tc_sage_layer(x, agg1, deg1, W_self1, W_neigh1, b1, relu=True)
    agg2, deg2 = _sc_segment_sum(h, src2, dst2, zrows, zdeg, ones)
    out = _tc_sage_layer(h, agg2, deg2, W_self2, W_neigh2, b2, relu=False)
    return out


# speedup vs baseline: 1.0007x; 1.0007x over previous
import functools

import jax
import jax.numpy as jnp
from jax import lax
from jax.experimental import pallas as pl
from jax.experimental.pallas import tpu as pltpu
from jax.experimental.pallas import tpu_sc as plsc

N = 10000
E = 320000
D = 128
DW = 8

NC = 2
NS = 16
NW = NC * NS
EPW = E // NW
B = 50
NB = EPW // B
NCH = 2
NB2 = NB // NCH
NBUF = 5
RPT = N // NS

_mesh = plsc.VectorSubcoreMesh(core_axis_name="c", subcore_axis_name="s")


@functools.partial(
    pl.kernel,
    out_type=(
        jax.ShapeDtypeStruct((NC, N, D), jnp.float32),
        jax.ShapeDtypeStruct((NC, N, DW), jnp.float32),
    ),
    mesh=_mesh,
    compiler_params=pltpu.CompilerParams(use_tc_tiling_on_sc=False),
    scratch_types=[
        pltpu.VMEM((NB2, B), jnp.int32),
        pltpu.VMEM((NB2, B), jnp.int32),
        pltpu.VMEM((B, D), jnp.float32),
        pltpu.VMEM((B, D), jnp.float32),
        pltpu.VMEM((B, D), jnp.float32),
        pltpu.VMEM((B, D), jnp.float32),
        pltpu.VMEM((B, D), jnp.float32),
        pltpu.VMEM((B, DW), jnp.float32),
        pltpu.VMEM_SHARED((N, D), jnp.float32),
        pltpu.VMEM_SHARED((N, DW), jnp.float32),
        pltpu.SemaphoreType.DMA,
        pltpu.SemaphoreType.DMA,
        pltpu.SemaphoreType.DMA,
        pltpu.SemaphoreType.DMA,
        pltpu.SemaphoreType.DMA,
        pltpu.SemaphoreType.DMA,
        pltpu.SemaphoreType.DMA,
        pltpu.SemaphoreType.DMA,
        pltpu.SemaphoreType.DMA,
        pltpu.SemaphoreType.DMA,
        pltpu.SemaphoreType.DMA,
    ],
)
def _sc_segment_sum(feat_hbm, src_hbm, dst_hbm, zrows_hbm, zdeg_hbm, ones_hbm,
                    agg_out, deg_out,
                    src_v, dst_v, r0, r1, r2, r3, r4, ones_v, agg_sh, deg_sh,
                    g0, g1, g2, g3, g4, s0, s1, s2, s3, s4, dsem):
    c = lax.axis_index("c")
    s = lax.axis_index("s")
    ring = (r0, r1, r2, r3, r4)
    gsems = (g0, g1, g2, g3, g4)
    ssems = (s0, s1, s2, s3, s4)

    pltpu.sync_copy(ones_hbm, ones_v)

    pltpu.sync_copy(zrows_hbm, agg_sh.at[pl.ds(s * RPT, RPT)])
    pltpu.sync_copy(zdeg_hbm, deg_sh.at[pl.ds(s * RPT, RPT)])
    plsc.subcore_barrier()

    def gather(j, b):
        pltpu.async_copy(feat_hbm.at[src_v.at[j]], ring[b], gsems[b])

    def gather_wait(j, b):
        pltpu.make_async_copy(feat_hbm.at[src_v.at[j]], ring[b], gsems[b]).wait()

    def scatter(j, b):
        pltpu.async_copy(ring[b], agg_sh.at[dst_v.at[j]], ssems[b], add=True)
        pltpu.async_copy(ones_v, deg_sh.at[dst_v.at[j]], dsem, add=True)

    def scatter_wait(j, b):
        pltpu.make_async_copy(ring[b], agg_sh.at[dst_v.at[j]], ssems[b]).wait()

    for ch in range(NCH):
        pltpu.sync_copy(src_hbm.at[c, s, ch], src_v)
        pltpu.sync_copy(dst_hbm.at[c, s, ch], dst_v)

        for b in range(NBUF):
            gather(b, b)

        def body(i, carry):
            j = NBUF * i
            for b in range(NBUF):
                gather_wait(j + b, b)
                scatter(j + b, b)
            for b in range(NBUF):
                scatter_wait(j + b, b)
                gather(j + NBUF + b, b)
            return carry

        lax.fori_loop(0, NB2 // NBUF - 1, body, 0)

        jlast = NB2 - NBUF
        for b in range(NBUF):
            gather_wait(jlast + b, b)
            scatter(jlast + b, b)
        for b in range(NBUF):
            scatter_wait(jlast + b, b)

        pltpu.make_async_copy(deg_out.at[c, pl.ds(0, NB2 * B)],
                              deg_sh.at[pl.ds(0, NB2 * B)], dsem).wait()
    plsc.subcore_barrier()

    pltpu.sync_copy(agg_sh.at[pl.ds(s * RPT, RPT)],
                    agg_out.at[c, pl.ds(s * RPT, RPT)])
    pltpu.sync_copy(deg_sh.at[pl.ds(s * RPT, RPT)],
                    deg_out.at[c, pl.ds(s * RPT, RPT)])


_R = 1000


def _tc_layer_body(relu, x_ref, agg_ref, deg_ref, ws_ref, wn_ref, b_ref,
                   o_ref):
    deg = deg_ref[0, :, 0] + deg_ref[1, :, 0]
    mean = (agg_ref[0] + agg_ref[1]) / jnp.maximum(deg, 1.0)[:, None]
    acc = jnp.dot(x_ref[...], ws_ref[...],
                  preferred_element_type=jnp.float32,
                  precision=lax.Precision.HIGHEST)
    acc = acc + jnp.dot(mean, wn_ref[...],
                        preferred_element_type=jnp.float32,
                        precision=lax.Precision.HIGHEST)
    acc = acc + b_ref[...]
    if relu:
        acc = jnp.maximum(acc, 0.0)
    o_ref[...] = acc


def _tc_sage_layer(x, agg, deg, W_self, W_neigh, b, relu):
    h = W_self.shape[1]
    return pl.pallas_call(
        functools.partial(_tc_layer_body, relu),
        grid=(N // _R,),
        in_specs=[
            pl.BlockSpec((_R, D), lambda i: (i, 0)),
            pl.BlockSpec((NC, _R, D), lambda i: (0, i, 0)),
            pl.BlockSpec((NC, _R, DW), lambda i: (0, i, 0)),
            pl.BlockSpec((D, h), lambda i: (0, 0)),
            pl.BlockSpec((D, h), lambda i: (0, 0)),
            pl.BlockSpec((1, h), lambda i: (0, 0)),
        ],
        out_specs=pl.BlockSpec((_R, h), lambda i: (i, 0)),
        out_shape=jax.ShapeDtypeStruct((N, h), jnp.float32),
    )(x, agg, deg, W_self, W_neigh, b.reshape(1, h))


def kernel(x, edge_index1, edge_index2, W_self1, W_neigh1, b1,
           W_self2, W_neigh2, b2):
    zrows = jnp.zeros((RPT, D), jnp.float32)
    zdeg = jnp.zeros((RPT, DW), jnp.float32)
    ones = jnp.ones((B, DW), jnp.float32)

    def edges(ei):
        src = ei[0].astype(jnp.int32).reshape(NC, NS, NCH, NB2, B)
        dst = ei[1].astype(jnp.int32).reshape(NC, NS, NCH, NB2, B)
        return src, dst

    src1, dst1 = edges(edge_index1)
    src2, dst2 = edges(edge_index2)

    agg1, deg1 = _sc_segment_sum(x, src1, dst1, zrows, zdeg, ones)
    h = _tc_sage_layer(x, agg1, deg1, W_self1, W_neigh1, b1, relu=True)
    agg2, deg2 = _sc_segment_sum(h, src2, dst2, zrows, zdeg, ones)
    out = _tc_sage_layer(h, agg2, deg2, W_self2, W_neigh2, b2, relu=False)
    return out


# speedup vs baseline: 1.0224x; 1.0217x over previous
import functools

import jax
import jax.numpy as jnp
from jax import lax
from jax.experimental import pallas as pl
from jax.experimental.pallas import tpu as pltpu
from jax.experimental.pallas import tpu_sc as plsc

N = 10000
E = 320000
D = 128
DW = 8

NC = 2
NS = 16
NW = NC * NS
EPW = E // NW
B = 50
NB = EPW // B
NCH = 2
NB2 = NB // NCH
NBUF = 5
RPT = N // NS

_mesh = plsc.VectorSubcoreMesh(core_axis_name="c", subcore_axis_name="s")


@functools.partial(
    pl.kernel,
    out_type=(
        jax.ShapeDtypeStruct((NC, N, D), jnp.float32),
        jax.ShapeDtypeStruct((NC, N, DW), jnp.float32),
    ),
    mesh=_mesh,
    compiler_params=pltpu.CompilerParams(use_tc_tiling_on_sc=False),
    scratch_types=[
        pltpu.VMEM((NB2, B), jnp.int32),
        pltpu.VMEM((NB2, B), jnp.int32),
        pltpu.VMEM((B, D), jnp.float32),
        pltpu.VMEM((B, D), jnp.float32),
        pltpu.VMEM((B, D), jnp.float32),
        pltpu.VMEM((B, D), jnp.float32),
        pltpu.VMEM((B, D), jnp.float32),
        pltpu.VMEM((B, DW), jnp.float32),
        pltpu.VMEM_SHARED((N, D), jnp.float32),
        pltpu.VMEM_SHARED((N, DW), jnp.float32),
        pltpu.SemaphoreType.DMA,
        pltpu.SemaphoreType.DMA,
        pltpu.SemaphoreType.DMA,
        pltpu.SemaphoreType.DMA,
        pltpu.SemaphoreType.DMA,
        pltpu.SemaphoreType.DMA,
        pltpu.SemaphoreType.DMA,
        pltpu.SemaphoreType.DMA,
        pltpu.SemaphoreType.DMA,
        pltpu.SemaphoreType.DMA,
        pltpu.SemaphoreType.DMA,
    ],
)
def _sc_segment_sum(feat_hbm, src_hbm, dst_hbm, zrows_hbm, zdeg_hbm, ones_hbm,
                    agg_out, deg_out,
                    src_v, dst_v, r0, r1, r2, r3, r4, ones_v, agg_sh, deg_sh,
                    g0, g1, g2, g3, g4, s0, s1, s2, s3, s4, dsem):
    c = lax.axis_index("c")
    s = lax.axis_index("s")
    ring = (r0, r1, r2, r3, r4)
    gsems = (g0, g1, g2, g3, g4)
    ssems = (s0, s1, s2, s3, s4)

    pltpu.sync_copy(ones_hbm, ones_v)

    pltpu.sync_copy(zrows_hbm, agg_sh.at[pl.ds(s * RPT, RPT)])
    pltpu.sync_copy(zdeg_hbm, deg_sh.at[pl.ds(s * RPT, RPT)])
    plsc.subcore_barrier()

    def gather(j, b):
        pltpu.async_copy(feat_hbm.at[src_v.at[j]], ring[b], gsems[b])

    def gather_wait(j, b):
        pltpu.make_async_copy(feat_hbm.at[src_v.at[j]], ring[b], gsems[b]).wait()

    def scatter(j, b):
        pltpu.async_copy(ring[b], agg_sh.at[dst_v.at[j]], ssems[b], add=True)
        pltpu.async_copy(ones_v, deg_sh.at[dst_v.at[j]], dsem, add=True)

    def scatter_wait(j, b):
        pltpu.make_async_copy(ring[b], agg_sh.at[dst_v.at[j]], ssems[b]).wait()

    for ch in range(NCH):
        pltpu.sync_copy(src_hbm.at[c, s, ch], src_v)
        pltpu.sync_copy(dst_hbm.at[c, s, ch], dst_v)

        for b in range(NBUF):
            gather(b, b)

        def body(i, carry):
            j = NBUF * i
            for b in range(NBUF):
                gather_wait(j + b, b)
                scatter(j + b, b)
            for b in range(NBUF):
                scatter_wait(j + b, b)
                gather(j + NBUF + b, b)
            return carry

        lax.fori_loop(0, NB2 // NBUF - 1, body, 0)

        jlast = NB2 - NBUF
        for b in range(NBUF):
            gather_wait(jlast + b, b)
            scatter(jlast + b, b)
        for b in range(NBUF):
            scatter_wait(jlast + b, b)

        pltpu.make_async_copy(deg_out.at[c, pl.ds(0, NB2 * B)],
                              deg_sh.at[pl.ds(0, NB2 * B)], dsem).wait()
    plsc.subcore_barrier()

    pltpu.sync_copy(agg_sh.at[pl.ds(s * RPT, RPT)],
                    agg_out.at[c, pl.ds(s * RPT, RPT)])
    pltpu.sync_copy(deg_sh.at[pl.ds(s * RPT, RPT)],
                    deg_out.at[c, pl.ds(s * RPT, RPT)])


_R = 1000


def _tc_self_body(x_ref, ws_ref, b_ref, o_ref):
    o_ref[...] = jnp.dot(x_ref[...], ws_ref[...],
                         preferred_element_type=jnp.float32,
                         precision=lax.Precision.HIGHEST) + b_ref[...]


def _tc_self(x, W_self, b):
    h = W_self.shape[1]
    return pl.pallas_call(
        _tc_self_body,
        grid=(N // _R,),
        in_specs=[
            pl.BlockSpec((_R, D), lambda i: (i, 0)),
            pl.BlockSpec((D, h), lambda i: (0, 0)),
            pl.BlockSpec((1, h), lambda i: (0, 0)),
        ],
        out_specs=pl.BlockSpec((_R, h), lambda i: (i, 0)),
        out_shape=jax.ShapeDtypeStruct((N, h), jnp.float32),
    )(x, W_self, b.reshape(1, h))


def _tc_combine_body(relu, self_ref, agg_ref, deg_ref, wn_ref, o_ref):
    deg = deg_ref[0, :, 0] + deg_ref[1, :, 0]
    mean = (agg_ref[0] + agg_ref[1]) / jnp.maximum(deg, 1.0)[:, None]
    acc = self_ref[...] + jnp.dot(mean, wn_ref[...],
                                  preferred_element_type=jnp.float32,
                                  precision=lax.Precision.HIGHEST)
    if relu:
        acc = jnp.maximum(acc, 0.0)
    o_ref[...] = acc


def _tc_combine(selfpart, agg, deg, W_neigh, relu):
    h = W_neigh.shape[1]
    return pl.pallas_call(
        functools.partial(_tc_combine_body, relu),
        grid=(N // _R,),
        in_specs=[
            pl.BlockSpec((_R, h), lambda i: (i, 0)),
            pl.BlockSpec((NC, _R, D), lambda i: (0, i, 0)),
            pl.BlockSpec((NC, _R, DW), lambda i: (0, i, 0)),
            pl.BlockSpec((D, h), lambda i: (0, 0)),
        ],
        out_specs=pl.BlockSpec((_R, h), lambda i: (i, 0)),
        out_shape=jax.ShapeDtypeStruct((N, h), jnp.float32),
    )(selfpart, agg, deg, W_neigh)


def kernel(x, edge_index1, edge_index2, W_self1, W_neigh1, b1,
           W_self2, W_neigh2, b2):
    zrows = jnp.zeros((RPT, D), jnp.float32)
    zdeg = jnp.zeros((RPT, DW), jnp.float32)
    ones = jnp.ones((B, DW), jnp.float32)

    def edges(ei):
        src = ei[0].astype(jnp.int32).reshape(NC, NS, NCH, NB2, B)
        dst = ei[1].astype(jnp.int32).reshape(NC, NS, NCH, NB2, B)
        return src, dst

    src1, dst1 = edges(edge_index1)
    src2, dst2 = edges(edge_index2)

    self1 = _tc_self(x, W_self1, b1)
    agg1, deg1 = _sc_segment_sum(x, src1, dst1, zrows, zdeg, ones)
    h = _tc_combine(self1, agg1, deg1, W_neigh1, relu=True)
    self2 = _tc_self(h, W_self2, b2)
    agg2, deg2 = _sc_segment_sum(h, src2, dst2, zrows, zdeg, ones)
    out = _tc_combine(self2, agg2, deg2, W_neigh2, relu=False)
    return out


# speedup vs baseline: 1.0355x; 1.0128x over previous
import functools

import jax
import jax.numpy as jnp
from jax import lax
from jax.experimental import pallas as pl
from jax.experimental.pallas import tpu as pltpu
from jax.experimental.pallas import tpu_sc as plsc

N = 10000
E = 320000
D = 128
DW = 8

NC = 2
NS = 16
NW = NC * NS
EPW = E // NW
B = 50
NB = EPW // B
NCH = 2
NB2 = NB // NCH
NBUF = 5
RPT = N // NS

_mesh = plsc.VectorSubcoreMesh(core_axis_name="c", subcore_axis_name="s")


@functools.partial(
    pl.kernel,
    out_type=(
        jax.ShapeDtypeStruct((NC, N, D), jnp.float32),
        jax.ShapeDtypeStruct((NC, N, DW), jnp.float32),
    ),
    mesh=_mesh,
    compiler_params=pltpu.CompilerParams(use_tc_tiling_on_sc=False),
    scratch_types=[
        pltpu.VMEM((NB2, B), jnp.int32),
        pltpu.VMEM((NB2, B), jnp.int32),
        pltpu.VMEM((B, D), jnp.float32),
        pltpu.VMEM((B, D), jnp.float32),
        pltpu.VMEM((B, D), jnp.float32),
        pltpu.VMEM((B, D), jnp.float32),
        pltpu.VMEM((B, D), jnp.float32),
        pltpu.VMEM((B, DW), jnp.float32),
        pltpu.VMEM_SHARED((N, D), jnp.float32),
        pltpu.VMEM_SHARED((N, DW), jnp.float32),
        pltpu.SemaphoreType.DMA,
        pltpu.SemaphoreType.DMA,
        pltpu.SemaphoreType.DMA,
        pltpu.SemaphoreType.DMA,
        pltpu.SemaphoreType.DMA,
        pltpu.SemaphoreType.DMA,
        pltpu.SemaphoreType.DMA,
        pltpu.SemaphoreType.DMA,
        pltpu.SemaphoreType.DMA,
        pltpu.SemaphoreType.DMA,
        pltpu.SemaphoreType.DMA,
    ],
)
def _sc_segment_sum(feat_hbm, src_hbm, dst_hbm, zrows_hbm, zdeg_hbm, ones_hbm,
                    agg_out, deg_out,
                    src_v, dst_v, r0, r1, r2, r3, r4, ones_v, agg_sh, deg_sh,
                    g0, g1, g2, g3, g4, s0, s1, s2, s3, s4, dsem):
    c = lax.axis_index("c")
    s = lax.axis_index("s")
    ring = (r0, r1, r2, r3, r4)
    gsems = (g0, g1, g2, g3, g4)
    ssems = (s0, s1, s2, s3, s4)

    pltpu.sync_copy(ones_hbm, ones_v)

    def gather(j, b):
        pltpu.async_copy(feat_hbm.at[src_v.at[j]], ring[b], gsems[b])

    def gather_wait(j, b):
        pltpu.make_async_copy(feat_hbm.at[src_v.at[j]], ring[b], gsems[b]).wait()

    def scatter(j, b):
        pltpu.async_copy(ring[b], agg_sh.at[dst_v.at[j]], ssems[b], add=True)
        pltpu.async_copy(ones_v, deg_sh.at[dst_v.at[j]], dsem, add=True)

    def scatter_wait(j, b):
        pltpu.make_async_copy(ring[b], agg_sh.at[dst_v.at[j]], ssems[b]).wait()

    pltpu.sync_copy(src_hbm.at[c, s, 0], src_v)
    pltpu.sync_copy(dst_hbm.at[c, s, 0], dst_v)
    for b in range(NBUF):
        gather(b, b)

    pltpu.sync_copy(zrows_hbm, agg_sh.at[pl.ds(s * RPT, RPT)])
    pltpu.sync_copy(zdeg_hbm, deg_sh.at[pl.ds(s * RPT, RPT)])
    plsc.subcore_barrier()

    for ch in range(NCH):
        if ch > 0:
            pltpu.sync_copy(src_hbm.at[c, s, ch], src_v)
            pltpu.sync_copy(dst_hbm.at[c, s, ch], dst_v)
            for b in range(NBUF):
                gather(b, b)

        def body(i, carry):
            j = NBUF * i
            for b in range(NBUF):
                gather_wait(j + b, b)
                scatter(j + b, b)
            for b in range(NBUF):
                scatter_wait(j + b, b)
                gather(j + NBUF + b, b)
            return carry

        lax.fori_loop(0, NB2 // NBUF - 1, body, 0)

        jlast = NB2 - NBUF
        for b in range(NBUF):
            gather_wait(jlast + b, b)
            scatter(jlast + b, b)
        for b in range(NBUF):
            scatter_wait(jlast + b, b)

        pltpu.make_async_copy(deg_out.at[c, pl.ds(0, NB2 * B)],
                              deg_sh.at[pl.ds(0, NB2 * B)], dsem).wait()
    plsc.subcore_barrier()

    pltpu.sync_copy(agg_sh.at[pl.ds(s * RPT, RPT)],
                    agg_out.at[c, pl.ds(s * RPT, RPT)])
    pltpu.sync_copy(deg_sh.at[pl.ds(s * RPT, RPT)],
                    deg_out.at[c, pl.ds(s * RPT, RPT)])


_R = 1000


def _tc_self_body(x_ref, ws_ref, b_ref, o_ref):
    o_ref[...] = jnp.dot(x_ref[...], ws_ref[...],
                         preferred_element_type=jnp.float32,
                         precision=lax.Precision.HIGHEST) + b_ref[...]


def _tc_self(x, W_self, b):
    h = W_self.shape[1]
    return pl.pallas_call(
        _tc_self_body,
        grid=(N // _R,),
        in_specs=[
            pl.BlockSpec((_R, D), lambda i: (i, 0)),
            pl.BlockSpec((D, h), lambda i: (0, 0)),
            pl.BlockSpec((1, h), lambda i: (0, 0)),
        ],
        out_specs=pl.BlockSpec((_R, h), lambda i: (i, 0)),
        out_shape=jax.ShapeDtypeStruct((N, h), jnp.float32),
    )(x, W_self, b.reshape(1, h))


def _tc_combine_body(relu, self_ref, agg_ref, deg_ref, wn_ref, o_ref):
    deg = deg_ref[0, :, 0] + deg_ref[1, :, 0]
    mean = (agg_ref[0] + agg_ref[1]) / jnp.maximum(deg, 1.0)[:, None]
    acc = self_ref[...] + jnp.dot(mean, wn_ref[...],
                                  preferred_element_type=jnp.float32,
                                  precision=lax.Precision.HIGHEST)
    if relu:
        acc = jnp.maximum(acc, 0.0)
    o_ref[...] = acc


def _tc_combine(selfpart, agg, deg, W_neigh, relu):
    h = W_neigh.shape[1]
    return pl.pallas_call(
        functools.partial(_tc_combine_body, relu),
        grid=(N // _R,),
        in_specs=[
            pl.BlockSpec((_R, h), lambda i: (i, 0)),
            pl.BlockSpec((NC, _R, D), lambda i: (0, i, 0)),
            pl.BlockSpec((NC, _R, DW), lambda i: (0, i, 0)),
            pl.BlockSpec((D, h), lambda i: (0, 0)),
        ],
        out_specs=pl.BlockSpec((_R, h), lambda i: (i, 0)),
        out_shape=jax.ShapeDtypeStruct((N, h), jnp.float32),
    )(selfpart, agg, deg, W_neigh)


def kernel(x, edge_index1, edge_index2, W_self1, W_neigh1, b1,
           W_self2, W_neigh2, b2):
    zrows = jnp.zeros((RPT, D), jnp.float32)
    zdeg = jnp.zeros((RPT, DW), jnp.float32)
    ones = jnp.ones((B, DW), jnp.float32)

    def edges(ei):
        src = ei[0].astype(jnp.int32).reshape(NC, NS, NCH, NB2, B)
        dst = ei[1].astype(jnp.int32).reshape(NC, NS, NCH, NB2, B)
        return src, dst

    src1, dst1 = edges(edge_index1)
    src2, dst2 = edges(edge_index2)

    self1 = _tc_self(x, W_self1, b1)
    agg1, deg1 = _sc_segment_sum(x, src1, dst1, zrows, zdeg, ones)
    h = _tc_combine(self1, agg1, deg1, W_neigh1, relu=True)
    self2 = _tc_self(h, W_self2, b2)
    agg2, deg2 = _sc_segment_sum(h, src2, dst2, zrows, zdeg, ones)
    out = _tc_combine(self2, agg2, deg2, W_neigh2, relu=False)
    return out


# speedup vs baseline: 1.0378x; 1.0023x over previous
import functools

import jax
import jax.numpy as jnp
from jax import lax
from jax.experimental import pallas as pl
from jax.experimental.pallas import tpu as pltpu
from jax.experimental.pallas import tpu_sc as plsc

N = 10000
E = 320000
D = 128
DW = 8

NC = 2
NS = 16
NW = NC * NS
EPW = E // NW
B = 50
NB = EPW // B
NCH = 2
NB2 = NB // NCH
NBUF = 5
RPT = N // NS

_mesh = plsc.VectorSubcoreMesh(core_axis_name="c", subcore_axis_name="s")


@functools.partial(
    pl.kernel,
    out_type=(
        jax.ShapeDtypeStruct((NC, N, D), jnp.float32),
        jax.ShapeDtypeStruct((NC, N, DW), jnp.float32),
    ),
    mesh=_mesh,
    compiler_params=pltpu.CompilerParams(use_tc_tiling_on_sc=False),
    scratch_types=[
        pltpu.VMEM((NB2, B), jnp.int32),
        pltpu.VMEM((NB2, B), jnp.int32),
        pltpu.VMEM((B, D), jnp.float32),
        pltpu.VMEM((B, D), jnp.float32),
        pltpu.VMEM((B, D), jnp.float32),
        pltpu.VMEM((B, D), jnp.float32),
        pltpu.VMEM((B, D), jnp.float32),
        pltpu.VMEM((B, DW), jnp.float32),
        pltpu.VMEM_SHARED((N, D), jnp.float32),
        pltpu.VMEM_SHARED((N, DW), jnp.float32),
        pltpu.SemaphoreType.DMA,
        pltpu.SemaphoreType.DMA,
        pltpu.SemaphoreType.DMA,
        pltpu.SemaphoreType.DMA,
        pltpu.SemaphoreType.DMA,
        pltpu.SemaphoreType.DMA,
        pltpu.SemaphoreType.DMA,
        pltpu.SemaphoreType.DMA,
        pltpu.SemaphoreType.DMA,
        pltpu.SemaphoreType.DMA,
        pltpu.SemaphoreType.DMA,
    ],
)
def _sc_segment_sum(feat_hbm, src_hbm, dst_hbm, zrows_hbm, zdeg_hbm, ones_hbm,
                    agg_out, deg_out,
                    src_v, dst_v, r0, r1, r2, r3, r4, ones_v, agg_sh, deg_sh,
                    g0, g1, g2, g3, g4, s0, s1, s2, s3, s4, dsem):
    c = lax.axis_index("c")
    s = lax.axis_index("s")
    ring = (r0, r1, r2, r3, r4)
    gsems = (g0, g1, g2, g3, g4)
    ssems = (s0, s1, s2, s3, s4)

    pltpu.sync_copy(ones_hbm, ones_v)

    def gather(j, b):
        pltpu.async_copy(feat_hbm.at[src_v.at[j]], ring[b], gsems[b])

    def gather_wait(j, b):
        pltpu.make_async_copy(feat_hbm.at[src_v.at[j]], ring[b], gsems[b]).wait()

    def scatter(j, b):
        pltpu.async_copy(ring[b], agg_sh.at[dst_v.at[j]], ssems[b], add=True)
        pltpu.async_copy(ones_v, deg_sh.at[dst_v.at[j]], dsem, add=True)

    def scatter_wait(j, b):
        pltpu.make_async_copy(ring[b], agg_sh.at[dst_v.at[j]], ssems[b]).wait()

    pltpu.sync_copy(src_hbm.at[c, s, 0], src_v)
    pltpu.sync_copy(dst_hbm.at[c, s, 0], dst_v)
    for b in range(NBUF):
        gather(b, b)

    pltpu.sync_copy(zrows_hbm, agg_sh.at[pl.ds(s * RPT, RPT)])
    pltpu.sync_copy(zdeg_hbm, deg_sh.at[pl.ds(s * RPT, RPT)])
    plsc.subcore_barrier()

    for ch in range(NCH):
        if ch > 0:
            pltpu.sync_copy(src_hbm.at[c, s, ch], src_v)
            pltpu.sync_copy(dst_hbm.at[c, s, ch], dst_v)
            for b in range(NBUF):
                gather(b, b)

        def body(i, carry):
            j = NBUF * i
            for b in range(NBUF):
                gather_wait(j + b, b)
                scatter(j + b, b)
            for b in range(NBUF):
                scatter_wait(j + b, b)
                gather(j + NBUF + b, b)
            return carry

        lax.fori_loop(0, NB2 // NBUF - 1, body, 0)

        jlast = NB2 - NBUF
        for b in range(NBUF):
            gather_wait(jlast + b, b)
            scatter(jlast + b, b)
        for b in range(NBUF):
            scatter_wait(jlast + b, b)

        pltpu.make_async_copy(deg_out.at[c, pl.ds(0, NB2 * B)],
                              deg_sh.at[pl.ds(0, NB2 * B)], dsem).wait()
    plsc.subcore_barrier()

    pltpu.async_copy(agg_sh.at[pl.ds(s * RPT, RPT)],
                     agg_out.at[c, pl.ds(s * RPT, RPT)], g0)
    pltpu.async_copy(deg_sh.at[pl.ds(s * RPT, RPT)],
                     deg_out.at[c, pl.ds(s * RPT, RPT)], g1)
    pltpu.make_async_copy(agg_sh.at[pl.ds(s * RPT, RPT)],
                          agg_out.at[c, pl.ds(s * RPT, RPT)], g0).wait()
    pltpu.make_async_copy(deg_sh.at[pl.ds(s * RPT, RPT)],
                          deg_out.at[c, pl.ds(s * RPT, RPT)], g1).wait()


_R = 1000


def _tc_self_body(x_ref, ws_ref, b_ref, o_ref):
    o_ref[...] = jnp.dot(x_ref[...], ws_ref[...],
                         preferred_element_type=jnp.float32,
                         precision=lax.Precision.HIGHEST) + b_ref[...]


def _tc_self(x, W_self, b):
    h = W_self.shape[1]
    return pl.pallas_call(
        _tc_self_body,
        grid=(N // _R,),
        in_specs=[
            pl.BlockSpec((_R, D), lambda i: (i, 0)),
            pl.BlockSpec((D, h), lambda i: (0, 0)),
            pl.BlockSpec((1, h), lambda i: (0, 0)),
        ],
        out_specs=pl.BlockSpec((_R, h), lambda i: (i, 0)),
        out_shape=jax.ShapeDtypeStruct((N, h), jnp.float32),
    )(x, W_self, b.reshape(1, h))


def _tc_combine_body(relu, self_ref, agg_ref, deg_ref, wn_ref, o_ref):
    deg = deg_ref[0, :, 0] + deg_ref[1, :, 0]
    mean = (agg_ref[0] + agg_ref[1]) / jnp.maximum(deg, 1.0)[:, None]
    acc = self_ref[...] + jnp.dot(mean, wn_ref[...],
                                  preferred_element_type=jnp.float32,
                                  precision=lax.Precision.HIGHEST)
    if relu:
        acc = jnp.maximum(acc, 0.0)
    o_ref[...] = acc


def _tc_combine(selfpart, agg, deg, W_neigh, relu):
    h = W_neigh.shape[1]
    return pl.pallas_call(
        functools.partial(_tc_combine_body, relu),
        grid=(N // _R,),
        in_specs=[
            pl.BlockSpec((_R, h), lambda i: (i, 0)),
            pl.BlockSpec((NC, _R, D), lambda i: (0, i, 0)),
            pl.BlockSpec((NC, _R, DW), lambda i: (0, i, 0)),
            pl.BlockSpec((D, h), lambda i: (0, 0)),
        ],
        out_specs=pl.BlockSpec((_R, h), lambda i: (i, 0)),
        out_shape=jax.ShapeDtypeStruct((N, h), jnp.float32),
    )(selfpart, agg, deg, W_neigh)


def kernel(x, edge_index1, edge_index2, W_self1, W_neigh1, b1,
           W_self2, W_neigh2, b2):
    zrows = jnp.zeros((RPT, D), jnp.float32)
    zdeg = jnp.zeros((RPT, DW), jnp.float32)
    ones = jnp.ones((B, DW), jnp.float32)

    def edges(ei):
        src = ei[0].astype(jnp.int32).reshape(NC, NS, NCH, NB2, B)
        dst = ei[1].astype(jnp.int32).reshape(NC, NS, NCH, NB2, B)
        return src, dst

    src1, dst1 = edges(edge_index1)
    src2, dst2 = edges(edge_index2)

    self1 = _tc_self(x, W_self1, b1)
    agg1, deg1 = _sc_segment_sum(x, src1, dst1, zrows, zdeg, ones)
    h = _tc_combine(self1, agg1, deg1, W_neigh1, relu=True)
    self2 = _tc_self(h, W_self2, b2)
    agg2, deg2 = _sc_segment_sum(h, src2, dst2, zrows, zdeg, ones)
    out = _tc_combine(self2, agg2, deg2, W_neigh2, relu=False)
    return out
